# baseline (speedup 1.0000x reference)
import functools

import jax
import jax.numpy as jnp
from jax import lax
from jax.experimental import pallas as pl
from jax.experimental.pallas import tpu as pltpu
from jax.experimental.pallas import tpu_sc as plsc

N = 10000
NP = 10112
DF = 128
B = 4096
NC, NS = 2, 16
NW = NC * NS
CH = 128
NCHUNK = 80
EPT = NCHUNK * CH
EP = NW * EPT
RPT = NP // NS
GPT = 2 * B // NW
GCH = GPT // CH
NCH0 = NCHUNK
NCH1 = NCHUNK

@functools.lru_cache(maxsize=None)
def _mesh():
    return plsc.VectorSubcoreMesh(
        core_axis_name="c", subcore_axis_name="s",
        num_cores=NC, num_subcores=NS)


def _deg_body(dsts, zer, out, dst_i, degloc):
    c = lax.axis_index("c")
    s = lax.axis_index("s")
    wid = c * NS + s
    pltpu.sync_copy(zer, degloc)
    pltpu.sync_copy(dsts.at[wid], dst_i)
    ones = jnp.ones((16,), jnp.float32)
    nstep = (NCH0 * CH // 16 if NCH0 == NCH1
             else jnp.where(c == 0, NCH0 * CH // 16, NCH1 * CH // 16))

    def step(i, carry):
        idx = dst_i[pl.ds(i * 16, 16)]
        plsc.addupdate_scatter(degloc, [idx], ones)
        return carry

    lax.fori_loop(0, nstep, step, 0)
    pltpu.sync_copy(degloc, out.at[wid])


@functools.lru_cache(maxsize=None)
def _sc_deg_k():
  return pl.kernel(
    _deg_body,
    out_type=jax.ShapeDtypeStruct((NW, NP), jnp.float32),
    mesh=_mesh(),
    scratch_types=[
        pltpu.VMEM((NCHMAX * CH,), jnp.int32),
        pltpu.VMEM((NP,), jnp.float32),
    ],
    compiler_params=pltpu.CompilerParams(needs_layout_passes=False),
  )


def _sc_deg(dsts_flat, zer_np):
    return _sc_deg_k()(dsts_flat, zer_np)


def _spmv_body(v_hbm, srcs, dsts, zer, out, src_i, dst_i, rows, acc, sem):
    c = lax.axis_index("c")
    s = lax.axis_index("s")
    wid = c * NS + s
    pltpu.sync_copy(zer, acc.at[pl.ds(s * RPT, RPT)])
    plsc.subcore_barrier()

    def chunk(j, carry):
        pltpu.async_copy(v_hbm.at[src_i.at[j]], rows, sem).wait()
        pltpu.sync_copy(rows, acc.at[dst_i.at[j]], add=True)
        return carry

    HB1 = NCHUNK // 2
    HB2 = NCHUNK - HB1
    pltpu.sync_copy(srcs.at[wid, pl.ds(0, HB1)], src_i.at[pl.ds(0, HB1)])
    pltpu.sync_copy(dsts.at[wid, pl.ds(0, HB1)], dst_i.at[pl.ds(0, HB1)])
    lax.fori_loop(0, HB1, chunk, 0)
    pltpu.sync_copy(srcs.at[wid, pl.ds(HB1, HB2)], src_i.at[pl.ds(0, HB2)])
    pltpu.sync_copy(dsts.at[wid, pl.ds(HB1, HB2)], dst_i.at[pl.ds(0, HB2)])
    lax.fori_loop(0, HB2, chunk, 0)
    plsc.subcore_barrier()
    pltpu.sync_copy(acc.at[pl.ds(s * RPT, RPT)], out.at[c, pl.ds(s * RPT, RPT)])


@functools.lru_cache(maxsize=None)
def _sc_spmv_k():
  return pl.kernel(
    _spmv_body,
    out_type=jax.ShapeDtypeStruct((NC, NP, DF), jnp.float32),
    mesh=_mesh(),
    scratch_types=[
        pltpu.VMEM((NCHUNK - NCHUNK // 2, CH), jnp.int32),
        pltpu.VMEM((NCHUNK - NCHUNK // 2, CH), jnp.int32),
        pltpu.VMEM((CH, DF), jnp.float32),
        pltpu.VMEM_SHARED((NP, DF), jnp.float32),
        pltpu.SemaphoreType.DMA,
    ],
  )


def _sc_spmv(v, srcs, dsts, zer):
    return _sc_spmv_k()(v, srcs, dsts, zer)


def _gather_body(q_hbm, idx_h, out, idx_v, rows, sem):
    c = lax.axis_index("c")
    s = lax.axis_index("s")
    wid = c * NS + s
    pltpu.sync_copy(idx_h.at[wid], idx_v)
    for j in range(GCH):
        pltpu.async_copy(q_hbm.at[idx_v.at[j]], rows, sem).wait()
        pltpu.sync_copy(rows, out.at[pl.ds(wid * GPT + j * CH, CH)])


@functools.lru_cache(maxsize=None)
def _sc_gather_k():
  return pl.kernel(
    _gather_body,
    out_type=jax.ShapeDtypeStruct((2 * B, DF), jnp.float32),
    mesh=_mesh(),
    scratch_types=[
        pltpu.VMEM((GCH, CH), jnp.int32),
        pltpu.VMEM((CH, DF), jnp.float32),
        pltpu.SemaphoreType.DMA,
    ],
  )


def _sc_gather(q, gidx):
    return _sc_gather_k()(q, gidx)


def _dis_col(dp):
    ones32 = jnp.ones((NW, 1), jnp.float32)
    deg = lax.dot_general(dp[...], ones32, (((0,), (0,)), ((), ())),
                          preferred_element_type=jnp.float32) + 1.0
    return lax.rsqrt(deg)


def _tc1_body(x_ref, w_ref, dp_ref, v1_ref):
    dis = _dis_col(dp_ref)
    u = jnp.dot(x_ref[...], w_ref[...], preferred_element_type=jnp.float32)
    v1_ref[...] = u * dis


def _tc1(xp, w1c, dp):
    return pl.pallas_call(
        _tc1_body,
        out_shape=jax.ShapeDtypeStruct((NP, DF), jnp.float32),
    )(xp, w1c, dp)


def _tc2_body(s1a, s1b, v1, dp, gac, btc, w2, v2s):
    dis = _dis_col(dp)
    p = (s1a[...] + s1b[...] + v1[...]) * dis
    m = jnp.sum(p, axis=0, keepdims=True) * (1.0 / N)
    v = jnp.sum(p * p, axis=0, keepdims=True) * (1.0 / N) - m * m
    h = jnp.maximum(gac[...] * (p - m) * lax.rsqrt(v + 1e-5) + btc[...], 0.0)
    v2 = jnp.dot(h, w2[...], preferred_element_type=jnp.float32)
    mask = lax.broadcasted_iota(jnp.int32, (NP, 1), 0) < N
    v2s[...] = jnp.where(mask, v2 * dis, 0.0)


def _tc2(s1a, s1b, v1, dp, gac, btc, w2blk):
    return pl.pallas_call(
        _tc2_body,
        out_shape=jax.ShapeDtypeStruct((NP, DF), jnp.float32),
    )(s1a, s1b, v1, dp, gac, btc, w2blk)


def _tc25_body(s2a, s2b, v2s, dp, qnb):
    dis = _dis_col(dp)
    qnb[...] = (s2a[...] + s2b[...] + v2s[...]) * dis


def _tc25(s2a, s2b, v2s, dp):
    return pl.pallas_call(
        _tc25_body,
        out_shape=jax.ShapeDtypeStruct((NP, DF), jnp.float32),
    )(s2a, s2b, v2s, dp)


def _tc3_body(g, b2g, b2s, wh1, ws1, ws2, wc, bc, wp, bp, link, gen):
    gu = g[0:B, :]
    gi = g[B:2 * B, :]
    genf = jnp.concatenate([gu[:, :64], gi[:, :64]], axis=1) + b2g[...]
    specf = jnp.concatenate([gu[:, 64:], gi[:, 64:]], axis=1) + b2s[...]
    h1 = jnp.tanh(jnp.dot(genf, wh1[...], preferred_element_type=jnp.float32))
    h2 = jnp.tanh(jnp.dot(specf, wh1[...], preferred_element_type=jnp.float32))
    zl = (jnp.dot(h1, ws1[...], preferred_element_type=jnp.float32)
          + jnp.dot(h2, ws2[...], preferred_element_type=jnp.float32))
    z = jax.nn.sigmoid(zl)
    fused = z * h1 + (1.0 - z) * h2
    gen[...] = jnp.dot(genf, wc[...], preferred_element_type=jnp.float32) + bc[...]
    link[...] = jnp.dot(fused, wp[...], preferred_element_type=jnp.float32) + bp[...]


def _tc3(g, b2g, b2s, wh1, ws1, ws2, wc, bc, wp, bp):
    return pl.pallas_call(
        _tc3_body,
        out_shape=(
            jax.ShapeDtypeStruct((B, 2), jnp.float32),
            jax.ShapeDtypeStruct((B, 4), jnp.float32),
        ),
    )(g, b2g, b2s, wh1, ws1, ws2, wc, bc, wp, bp)


NCHMAX = max(NCH0, NCH1)


def _edge_layout(ix):
    f = jnp.int32
    e = ix.shape[0]
    n0 = NS * NCH0 * CH
    n1 = NS * NCH1 * CH
    a = ix[:n0].reshape(NS, NCH0, CH)
    a = jnp.concatenate(
        [a, jnp.full((NS, NCHMAX - NCH0, CH), N, f)], axis=1)
    b = jnp.concatenate([ix[n0:], jnp.full((n1 - (e - n0),), N, f)])
    b = b.reshape(NS, NCH1, CH)
    b = jnp.concatenate(
        [b, jnp.full((NS, NCHMAX - NCH1, CH), N, f)], axis=1)
    return jnp.concatenate([a, b], axis=0)


def kernel(x, edge_index, users_tensor, items_tensor, W1a, b1a, ga, bta,
           W2a, b2a, W1b, b1b, gb, btb, W2b, b2b, Wh1, Wh2, Ws, Wc, bc,
           Wp, bp):
    f32 = jnp.float32
    e = edge_index.shape[1]
    padi = jnp.full((EP - e,), N, jnp.int32)
    srcs = jnp.concatenate([edge_index[0], padi]).reshape(NW, NCHUNK, CH)
    dsts = jnp.concatenate([edge_index[1], padi]).reshape(NW, NCHUNK, CH)
    xp = jnp.zeros((NP, DF), f32).at[:N, :].set(x)
    zer128 = jnp.zeros((RPT, DF), f32)
    zer_np = jnp.zeros((NP,), f32)

    dp = _sc_deg(dsts.reshape(NW, NCHMAX * CH), zer_np)


    w1c = jnp.concatenate([W1a, W1b], axis=1)
    v1 = _tc1(xp, w1c, dp)

    s1 = _sc_spmv(v1, srcs, dsts, zer128)

    z64 = jnp.zeros((64, 64), f32)
    w2blk = jnp.concatenate(
        [jnp.concatenate([W2a, z64], axis=1),
         jnp.concatenate([z64, W2b], axis=1)], axis=0)
    gac = jnp.concatenate([ga, gb]).reshape(1, DF)
    btc = jnp.concatenate([bta, btb]).reshape(1, DF)
    v2s = _tc2(s1[0], s1[1], v1, dp, gac, btc, w2blk)

    s2 = _sc_spmv(v2s, srcs, dsts, zer128)

    qnb = _tc25(s2[0], s2[1], v2s, dp)

    gidx = jnp.concatenate([users_tensor, items_tensor]).reshape(NW, GCH, CH)
    g = _sc_gather(qnb, gidx)

    b2g = jnp.concatenate([b2a, b2a]).reshape(1, DF)
    b2s = jnp.concatenate([b2b, b2b]).reshape(1, DF)
    link_out, gen_out = _tc3(
        g, b2g, b2s, Wh1, Ws[:DF, :], Ws[DF:, :], Wc, bc.reshape(1, 4),
        Wp, bp.reshape(1, 2))
    return (link_out, gen_out)


# speedup vs baseline: 2.6729x; 2.6729x over previous
import functools

import jax
import jax.numpy as jnp
from jax import lax
from jax.experimental import pallas as pl
from jax.experimental.pallas import tpu as pltpu
from jax.experimental.pallas import tpu_sc as plsc

N = 10000
NP = 10112
DF = 128
B = 4096
NC, NS = 2, 16
NW = NC * NS
CH = 128
NCHUNK = 80
EPT = NCHUNK * CH
EP = NW * EPT
RPT = NP // NS
GPT = 2 * B // NW
GCH = GPT // CH
NCH0 = NCHUNK
NCH1 = NCHUNK

@functools.lru_cache(maxsize=None)
def _mesh():
    return plsc.VectorSubcoreMesh(
        core_axis_name="c", subcore_axis_name="s",
        num_cores=NC, num_subcores=NS)


def _deg_body(dsts, zer, out, dst_i, degloc):
    c = lax.axis_index("c")
    s = lax.axis_index("s")
    wid = c * NS + s
    pltpu.sync_copy(zer, degloc)
    pltpu.sync_copy(dsts.at[wid], dst_i)
    ones = jnp.ones((16,), jnp.float32)
    nstep = (NCH0 * CH // 16 if NCH0 == NCH1
             else jnp.where(c == 0, NCH0 * CH // 16, NCH1 * CH // 16))

    def step(i, carry):
        idx = dst_i[pl.ds(i * 16, 16)]
        plsc.addupdate_scatter(degloc, [idx], ones)
        return carry

    lax.fori_loop(0, nstep, step, 0)
    pltpu.sync_copy(degloc, out.at[wid])


@functools.lru_cache(maxsize=None)
def _sc_deg_k():
  return pl.kernel(
    _deg_body,
    out_type=jax.ShapeDtypeStruct((NW, NP), jnp.float32),
    mesh=_mesh(),
    scratch_types=[
        pltpu.VMEM((NCHMAX * CH,), jnp.int32),
        pltpu.VMEM((NP,), jnp.float32),
    ],
    compiler_params=pltpu.CompilerParams(needs_layout_passes=False),
  )


def _sc_deg(dsts_flat, zer_np):
    return _sc_deg_k()(dsts_flat, zer_np)


def _spmv_body(v_hbm, srcs, dsts, zer, out, src_i, dst_i, rows, acc, sem):
    c = lax.axis_index("c")
    s = lax.axis_index("s")
    wid = c * NS + s
    pltpu.sync_copy(zer, acc.at[pl.ds(s * RPT, RPT)])
    plsc.subcore_barrier()

    def chunk(j, carry):
        pltpu.async_copy(v_hbm.at[src_i.at[j]], rows, sem).wait()
        pltpu.sync_copy(rows, acc.at[dst_i.at[j]], add=True)
        return carry

    HB1 = NCHUNK // 2
    HB2 = NCHUNK - HB1
    pltpu.sync_copy(srcs.at[wid, pl.ds(0, HB1)], src_i.at[pl.ds(0, HB1)])
    pltpu.sync_copy(dsts.at[wid, pl.ds(0, HB1)], dst_i.at[pl.ds(0, HB1)])
    lax.fori_loop(0, HB1, chunk, 0)
    pltpu.sync_copy(srcs.at[wid, pl.ds(HB1, HB2)], src_i.at[pl.ds(0, HB2)])
    pltpu.sync_copy(dsts.at[wid, pl.ds(HB1, HB2)], dst_i.at[pl.ds(0, HB2)])
    lax.fori_loop(0, HB2, chunk, 0)
    plsc.subcore_barrier()
    pltpu.sync_copy(acc.at[pl.ds(s * RPT, RPT)], out.at[c, pl.ds(s * RPT, RPT)])


@functools.lru_cache(maxsize=None)
def _sc_spmv_k():
  return pl.kernel(
    _spmv_body,
    out_type=jax.ShapeDtypeStruct((NC, NP, DF), jnp.float32),
    mesh=_mesh(),
    scratch_types=[
        pltpu.VMEM((NCHUNK - NCHUNK // 2, CH), jnp.int32),
        pltpu.VMEM((NCHUNK - NCHUNK // 2, CH), jnp.int32),
        pltpu.VMEM((CH, DF), jnp.float32),
        pltpu.VMEM_SHARED((NP, DF), jnp.float32),
        pltpu.SemaphoreType.DMA,
    ],
  )


def _sc_spmv(v, srcs, dsts, zer):
    return _sc_spmv_k()(v, srcs, dsts, zer)


def _gather_body(q_hbm, idx_h, out, idx_v, rows, sem):
    c = lax.axis_index("c")
    s = lax.axis_index("s")
    wid = c * NS + s
    pltpu.sync_copy(idx_h.at[wid], idx_v)
    for j in range(GCH):
        pltpu.async_copy(q_hbm.at[idx_v.at[j]], rows, sem).wait()
        pltpu.sync_copy(rows, out.at[pl.ds(wid * GPT + j * CH, CH)])


@functools.lru_cache(maxsize=None)
def _sc_gather_k():
  return pl.kernel(
    _gather_body,
    out_type=jax.ShapeDtypeStruct((2 * B, DF), jnp.float32),
    mesh=_mesh(),
    scratch_types=[
        pltpu.VMEM((GCH, CH), jnp.int32),
        pltpu.VMEM((CH, DF), jnp.float32),
        pltpu.SemaphoreType.DMA,
    ],
  )


def _sc_gather(q, gidx):
    return _sc_gather_k()(q, gidx)


def _dis_col(dp):
    ones32 = jnp.ones((NW, 1), jnp.float32)
    deg = lax.dot_general(dp[...], ones32, (((0,), (0,)), ((), ())),
                          preferred_element_type=jnp.float32) + 1.0
    return lax.rsqrt(deg)


def _tc1_body(x_ref, w_ref, dp_ref, v1_ref):
    dis = _dis_col(dp_ref)
    u = jnp.dot(x_ref[...], w_ref[...], preferred_element_type=jnp.float32)
    v1_ref[...] = u * dis


def _tc1(xp, w1c, dp):
    return pl.pallas_call(
        _tc1_body,
        out_shape=jax.ShapeDtypeStruct((NP, DF), jnp.float32),
    )(xp, w1c, dp)


def _tc2_body(s1a, s1b, v1, dp, gac, btc, w2, v2s):
    dis = _dis_col(dp)
    p = (s1a[...] + s1b[...] + v1[...]) * dis
    m = jnp.sum(p, axis=0, keepdims=True) * (1.0 / N)
    v = jnp.sum(p * p, axis=0, keepdims=True) * (1.0 / N) - m * m
    h = jnp.maximum(gac[...] * (p - m) * lax.rsqrt(v + 1e-5) + btc[...], 0.0)
    v2 = jnp.dot(h, w2[...], preferred_element_type=jnp.float32)
    mask = lax.broadcasted_iota(jnp.int32, (NP, 1), 0) < N
    v2s[...] = jnp.where(mask, v2 * dis, 0.0)


def _tc2(s1a, s1b, v1, dp, gac, btc, w2blk):
    return pl.pallas_call(
        _tc2_body,
        out_shape=jax.ShapeDtypeStruct((NP, DF), jnp.float32),
    )(s1a, s1b, v1, dp, gac, btc, w2blk)


def _tc25_body(s2a, s2b, v2s, dp, qnb):
    dis = _dis_col(dp)
    qnb[...] = (s2a[...] + s2b[...] + v2s[...]) * dis


def _tc25(s2a, s2b, v2s, dp):
    return pl.pallas_call(
        _tc25_body,
        out_shape=jax.ShapeDtypeStruct((NP, DF), jnp.float32),
    )(s2a, s2b, v2s, dp)


def _tc3_body(g, b2g, b2s, wh1, ws1, ws2, wc, bc, wp, bp, link, gen):
    gu = g[0:B, :]
    gi = g[B:2 * B, :]
    genf = jnp.concatenate([gu[:, :64], gi[:, :64]], axis=1) + b2g[...]
    specf = jnp.concatenate([gu[:, 64:], gi[:, 64:]], axis=1) + b2s[...]
    h1 = jnp.tanh(jnp.dot(genf, wh1[...], preferred_element_type=jnp.float32))
    h2 = jnp.tanh(jnp.dot(specf, wh1[...], preferred_element_type=jnp.float32))
    zl = (jnp.dot(h1, ws1[...], preferred_element_type=jnp.float32)
          + jnp.dot(h2, ws2[...], preferred_element_type=jnp.float32))
    z = jax.nn.sigmoid(zl)
    fused = z * h1 + (1.0 - z) * h2
    gen[...] = jnp.dot(genf, wc[...], preferred_element_type=jnp.float32) + bc[...]
    link[...] = jnp.dot(fused, wp[...], preferred_element_type=jnp.float32) + bp[...]


def _tc3(g, b2g, b2s, wh1, ws1, ws2, wc, bc, wp, bp):
    return pl.pallas_call(
        _tc3_body,
        out_shape=(
            jax.ShapeDtypeStruct((B, 2), jnp.float32),
            jax.ShapeDtypeStruct((B, 4), jnp.float32),
        ),
    )(g, b2g, b2s, wh1, ws1, ws2, wc, bc, wp, bp)


NCHMAX = max(NCH0, NCH1)


def _edge_layout(ix):
    f = jnp.int32
    e = ix.shape[0]
    n0 = NS * NCH0 * CH
    n1 = NS * NCH1 * CH
    a = ix[:n0].reshape(NS, NCH0, CH)
    a = jnp.concatenate(
        [a, jnp.full((NS, NCHMAX - NCH0, CH), N, f)], axis=1)
    b = jnp.concatenate([ix[n0:], jnp.full((n1 - (e - n0),), N, f)])
    b = b.reshape(NS, NCH1, CH)
    b = jnp.concatenate(
        [b, jnp.full((NS, NCHMAX - NCH1, CH), N, f)], axis=1)
    return jnp.concatenate([a, b], axis=0)


def kernel(x, edge_index, users_tensor, items_tensor, W1a, b1a, ga, bta,
           W2a, b2a, W1b, b1b, gb, btb, W2b, b2b, Wh1, Wh2, Ws, Wc, bc,
           Wp, bp):
    f32 = jnp.float32
    e = edge_index.shape[1]
    padi = N + (jnp.arange(EP - e, dtype=jnp.int32) % (NP - N))
    srcs = jnp.concatenate([edge_index[0], padi]).reshape(NW, NCHUNK, CH)
    dsts = jnp.concatenate([edge_index[1], padi]).reshape(NW, NCHUNK, CH)
    xp = jnp.zeros((NP, DF), f32).at[:N, :].set(x)
    zer128 = jnp.zeros((RPT, DF), f32)
    zer_np = jnp.zeros((NP,), f32)

    dp = _sc_deg(dsts.reshape(NW, NCHMAX * CH), zer_np)


    w1c = jnp.concatenate([W1a, W1b], axis=1)
    v1 = _tc1(xp, w1c, dp)

    s1 = _sc_spmv(v1, srcs, dsts, zer128)

    z64 = jnp.zeros((64, 64), f32)
    w2blk = jnp.concatenate(
        [jnp.concatenate([W2a, z64], axis=1),
         jnp.concatenate([z64, W2b], axis=1)], axis=0)
    gac = jnp.concatenate([ga, gb]).reshape(1, DF)
    btc = jnp.concatenate([bta, btb]).reshape(1, DF)
    v2s = _tc2(s1[0], s1[1], v1, dp, gac, btc, w2blk)

    s2 = _sc_spmv(v2s, srcs, dsts, zer128)

    qnb = _tc25(s2[0], s2[1], v2s, dp)

    gidx = jnp.concatenate([users_tensor, items_tensor]).reshape(NW, GCH, CH)
    g = _sc_gather(qnb, gidx)

    b2g = jnp.concatenate([b2a, b2a]).reshape(1, DF)
    b2s = jnp.concatenate([b2b, b2b]).reshape(1, DF)
    link_out, gen_out = _tc3(
        g, b2g, b2s, Wh1, Ws[:DF, :], Ws[DF:, :], Wc, bc.reshape(1, 4),
        Wp, bp.reshape(1, 2))
    return (link_out, gen_out)


# speedup vs baseline: 3.7626x; 1.4077x over previous
import functools

import jax
import jax.numpy as jnp
from jax import lax
from jax.experimental import pallas as pl
from jax.experimental.pallas import tpu as pltpu
from jax.experimental.pallas import tpu_sc as plsc

N = 10000
NP = 10112
DF = 128
B = 4096
NC, NS = 2, 16
NW = NC * NS
CH = 128
NCHUNK = 80
EPT = NCHUNK * CH
EP = NW * EPT
RPT = NP // NS
GPT = 2 * B // NW
GCH = GPT // CH
NCH0 = NCHUNK
NCH1 = NCHUNK

@functools.lru_cache(maxsize=None)
def _mesh():
    return plsc.VectorSubcoreMesh(
        core_axis_name="c", subcore_axis_name="s",
        num_cores=NC, num_subcores=NS)


def _deg_body(dsts, zer, out, dst_i, degloc):
    c = lax.axis_index("c")
    s = lax.axis_index("s")
    wid = c * NS + s
    pltpu.sync_copy(zer, degloc)
    pltpu.sync_copy(dsts.at[wid], dst_i)
    ones = jnp.ones((16,), jnp.float32)
    nstep = (NCH0 * CH // 16 if NCH0 == NCH1
             else jnp.where(c == 0, NCH0 * CH // 16, NCH1 * CH // 16))

    def step(i, carry):
        idx = dst_i[pl.ds(i * 16, 16)]
        plsc.addupdate_scatter(degloc, [idx], ones)
        return carry

    lax.fori_loop(0, nstep, step, 0)
    pltpu.sync_copy(degloc, out.at[wid])


@functools.lru_cache(maxsize=None)
def _sc_deg_k():
  return pl.kernel(
    _deg_body,
    out_type=jax.ShapeDtypeStruct((NW, NP), jnp.float32),
    mesh=_mesh(),
    scratch_types=[
        pltpu.VMEM((NCHMAX * CH,), jnp.int32),
        pltpu.VMEM((NP,), jnp.float32),
    ],
    compiler_params=pltpu.CompilerParams(needs_layout_passes=False),
  )


def _sc_deg(dsts_flat, zer_np):
    return _sc_deg_k()(dsts_flat, zer_np)


def _spmv_body(v_hbm, srcs, dsts, zer, out, src_i, dst_i, rows, acc,
               sem0, sem1):
    c = lax.axis_index("c")
    s = lax.axis_index("s")
    wid = c * NS + s
    pltpu.sync_copy(zer, acc.at[pl.ds(s * RPT, RPT)])
    plsc.subcore_barrier()

    sems = (sem0, sem1)
    HB = NCHUNK // 2
    for h in range(2):
        pltpu.sync_copy(srcs.at[wid, pl.ds(h * HB, HB)], src_i)
        pltpu.sync_copy(dsts.at[wid, pl.ds(h * HB, HB)], dst_i)
        pltpu.async_copy(v_hbm.at[src_i.at[0]], rows.at[0], sem0)
        pltpu.async_copy(v_hbm.at[src_i.at[1]], rows.at[1], sem1)

        def chunk(jj, carry):
            for b in range(2):
                j = jj * 2 + b
                pltpu.make_async_copy(v_hbm.at[src_i.at[j]], rows.at[b],
                                      sems[b]).wait()
                pltpu.sync_copy(rows.at[b], acc.at[dst_i.at[j]], add=True)

                @pl.when(j + 2 < HB)
                def _():
                    pltpu.async_copy(v_hbm.at[src_i.at[j + 2]], rows.at[b],
                                     sems[b])
            return carry

        lax.fori_loop(0, HB // 2, chunk, 0)
    plsc.subcore_barrier()
    pltpu.sync_copy(acc.at[pl.ds(s * RPT, RPT)], out.at[c, pl.ds(s * RPT, RPT)])


@functools.lru_cache(maxsize=None)
def _sc_spmv_k():
  return pl.kernel(
    _spmv_body,
    out_type=jax.ShapeDtypeStruct((NC, NP, DF), jnp.float32),
    mesh=_mesh(),
    scratch_types=[
        pltpu.VMEM((NCHUNK // 2, CH), jnp.int32),
        pltpu.VMEM((NCHUNK // 2, CH), jnp.int32),
        pltpu.VMEM((2, CH, DF), jnp.float32),
        pltpu.VMEM_SHARED((NP, DF), jnp.float32),
        pltpu.SemaphoreType.DMA,
        pltpu.SemaphoreType.DMA,
    ],
  )


def _sc_spmv(v, srcs, dsts, zer):
    return _sc_spmv_k()(v, srcs, dsts, zer)


def _gather_body(q_hbm, idx_h, out, idx_v, rows, sem):
    c = lax.axis_index("c")
    s = lax.axis_index("s")
    wid = c * NS + s
    pltpu.sync_copy(idx_h.at[wid], idx_v)
    for j in range(GCH):
        pltpu.async_copy(q_hbm.at[idx_v.at[j]], rows, sem).wait()
        pltpu.sync_copy(rows, out.at[pl.ds(wid * GPT + j * CH, CH)])


@functools.lru_cache(maxsize=None)
def _sc_gather_k():
  return pl.kernel(
    _gather_body,
    out_type=jax.ShapeDtypeStruct((2 * B, DF), jnp.float32),
    mesh=_mesh(),
    scratch_types=[
        pltpu.VMEM((GCH, CH), jnp.int32),
        pltpu.VMEM((CH, DF), jnp.float32),
        pltpu.SemaphoreType.DMA,
    ],
  )


def _sc_gather(q, gidx):
    return _sc_gather_k()(q, gidx)


def _dis_col(dp):
    ones32 = jnp.ones((NW, 1), jnp.float32)
    deg = lax.dot_general(dp[...], ones32, (((0,), (0,)), ((), ())),
                          preferred_element_type=jnp.float32) + 1.0
    return lax.rsqrt(deg)


def _tc1_body(x_ref, w_ref, dp_ref, v1_ref):
    dis = _dis_col(dp_ref)
    u = jnp.dot(x_ref[...], w_ref[...], preferred_element_type=jnp.float32)
    v1_ref[...] = u * dis


def _tc1(xp, w1c, dp):
    return pl.pallas_call(
        _tc1_body,
        out_shape=jax.ShapeDtypeStruct((NP, DF), jnp.float32),
    )(xp, w1c, dp)


def _tc2_body(s1a, s1b, v1, dp, gac, btc, w2, v2s):
    dis = _dis_col(dp)
    p = (s1a[...] + s1b[...] + v1[...]) * dis
    m = jnp.sum(p, axis=0, keepdims=True) * (1.0 / N)
    v = jnp.sum(p * p, axis=0, keepdims=True) * (1.0 / N) - m * m
    h = jnp.maximum(gac[...] * (p - m) * lax.rsqrt(v + 1e-5) + btc[...], 0.0)
    v2 = jnp.dot(h, w2[...], preferred_element_type=jnp.float32)
    mask = lax.broadcasted_iota(jnp.int32, (NP, 1), 0) < N
    v2s[...] = jnp.where(mask, v2 * dis, 0.0)


def _tc2(s1a, s1b, v1, dp, gac, btc, w2blk):
    return pl.pallas_call(
        _tc2_body,
        out_shape=jax.ShapeDtypeStruct((NP, DF), jnp.float32),
    )(s1a, s1b, v1, dp, gac, btc, w2blk)


def _tc25_body(s2a, s2b, v2s, dp, qnb):
    dis = _dis_col(dp)
    qnb[...] = (s2a[...] + s2b[...] + v2s[...]) * dis


def _tc25(s2a, s2b, v2s, dp):
    return pl.pallas_call(
        _tc25_body,
        out_shape=jax.ShapeDtypeStruct((NP, DF), jnp.float32),
    )(s2a, s2b, v2s, dp)


def _tc3_body(g, b2g, b2s, wh1, ws1, ws2, wc, bc, wp, bp, link, gen):
    gu = g[0:B, :]
    gi = g[B:2 * B, :]
    genf = jnp.concatenate([gu[:, :64], gi[:, :64]], axis=1) + b2g[...]
    specf = jnp.concatenate([gu[:, 64:], gi[:, 64:]], axis=1) + b2s[...]
    h1 = jnp.tanh(jnp.dot(genf, wh1[...], preferred_element_type=jnp.float32))
    h2 = jnp.tanh(jnp.dot(specf, wh1[...], preferred_element_type=jnp.float32))
    zl = (jnp.dot(h1, ws1[...], preferred_element_type=jnp.float32)
          + jnp.dot(h2, ws2[...], preferred_element_type=jnp.float32))
    z = jax.nn.sigmoid(zl)
    fused = z * h1 + (1.0 - z) * h2
    gen[...] = jnp.dot(genf, wc[...], preferred_element_type=jnp.float32) + bc[...]
    link[...] = jnp.dot(fused, wp[...], preferred_element_type=jnp.float32) + bp[...]


def _tc3(g, b2g, b2s, wh1, ws1, ws2, wc, bc, wp, bp):
    return pl.pallas_call(
        _tc3_body,
        out_shape=(
            jax.ShapeDtypeStruct((B, 2), jnp.float32),
            jax.ShapeDtypeStruct((B, 4), jnp.float32),
        ),
    )(g, b2g, b2s, wh1, ws1, ws2, wc, bc, wp, bp)


NCHMAX = max(NCH0, NCH1)


def _edge_layout(ix):
    f = jnp.int32
    e = ix.shape[0]
    n0 = NS * NCH0 * CH
    n1 = NS * NCH1 * CH
    a = ix[:n0].reshape(NS, NCH0, CH)
    a = jnp.concatenate(
        [a, jnp.full((NS, NCHMAX - NCH0, CH), N, f)], axis=1)
    b = jnp.concatenate([ix[n0:], jnp.full((n1 - (e - n0),), N, f)])
    b = b.reshape(NS, NCH1, CH)
    b = jnp.concatenate(
        [b, jnp.full((NS, NCHMAX - NCH1, CH), N, f)], axis=1)
    return jnp.concatenate([a, b], axis=0)


def kernel(x, edge_index, users_tensor, items_tensor, W1a, b1a, ga, bta,
           W2a, b2a, W1b, b1b, gb, btb, W2b, b2b, Wh1, Wh2, Ws, Wc, bc,
           Wp, bp):
    f32 = jnp.float32
    e = edge_index.shape[1]
    padi = N + (jnp.arange(EP - e, dtype=jnp.int32) % (NP - N))
    srcs = jnp.concatenate([edge_index[0], padi]).reshape(NW, NCHUNK, CH)
    dsts = jnp.concatenate([edge_index[1], padi]).reshape(NW, NCHUNK, CH)
    xp = jnp.zeros((NP, DF), f32).at[:N, :].set(x)
    zer128 = jnp.zeros((RPT, DF), f32)
    zer_np = jnp.zeros((NP,), f32)

    dp = _sc_deg(dsts.reshape(NW, NCHMAX * CH), zer_np)


    w1c = jnp.concatenate([W1a, W1b], axis=1)
    v1 = _tc1(xp, w1c, dp)

    s1 = _sc_spmv(v1, srcs, dsts, zer128)

    z64 = jnp.zeros((64, 64), f32)
    w2blk = jnp.concatenate(
        [jnp.concatenate([W2a, z64], axis=1),
         jnp.concatenate([z64, W2b], axis=1)], axis=0)
    gac = jnp.concatenate([ga, gb]).reshape(1, DF)
    btc = jnp.concatenate([bta, btb]).reshape(1, DF)
    v2s = _tc2(s1[0], s1[1], v1, dp, gac, btc, w2blk)

    s2 = _sc_spmv(v2s, srcs, dsts, zer128)

    qnb = _tc25(s2[0], s2[1], v2s, dp)

    gidx = jnp.concatenate([users_tensor, items_tensor]).reshape(NW, GCH, CH)
    g = _sc_gather(qnb, gidx)

    b2g = jnp.concatenate([b2a, b2a]).reshape(1, DF)
    b2s = jnp.concatenate([b2b, b2b]).reshape(1, DF)
    link_out, gen_out = _tc3(
        g, b2g, b2s, Wh1, Ws[:DF, :], Ws[DF:, :], Wc, bc.reshape(1, 4),
        Wp, bp.reshape(1, 2))
    return (link_out, gen_out)
